# Initial kernel scaffold; baseline (speedup 1.0000x reference)
import jax
import jax.numpy as jnp
from jax.experimental import pallas as pl


def kernel(x, wave_w, wave_b, kernel_w, kernel_b, out_w):
    raise NotImplementedError("write your pallas kernel here")


# speedup vs baseline: 20.0371x; 20.0371x over previous
import jax
import jax.numpy as jnp
from jax import lax
from jax.experimental import pallas as pl

L = 2048
C = 1024
H = 16
K = 64
D = C // H
S = 17
HALF_S = 8
MAX_FREQ = 16.0
MIN_FREQ = 1.0
MAX_RECEPTIVE = 128.0

BLK = 256
WIN = 768


def _sigmoid(t):
    return 1.0 / (1.0 + jnp.exp(-t))


def _tanh(t):
    return 2.0 / (1.0 + jnp.exp(-2.0 * t)) - 1.0


def _silu(t):
    return t * _sigmoid(t)


def _dot_t(a, b):
    return lax.dot_general(
        a.astype(jnp.bfloat16), b.astype(jnp.bfloat16), (((1,), (1,)), ((), ())),
        preferred_element_type=jnp.float32,
    )


def _gather_kernel(xb_ref, xh_ref, wwf_ref, wwp_ref, wb_ref, kw_ref, kb_ref,
                   hid_ref):
    i = pl.program_id(0)
    l0 = i * BLK
    xb = xb_ref[...]

    xb_r = xb.astype(jnp.bfloat16).astype(jnp.float32)
    wwf_r = wwf_ref[0].astype(jnp.bfloat16).astype(jnp.float32)
    wwp_r = wwp_ref[0].astype(jnp.bfloat16).astype(jnp.float32)
    wave_f = _silu(jnp.sum(xb_r * wwf_r, axis=1, keepdims=True)
                   + wb_ref[0, 0, 0])
    wave_p = _silu(jnp.sum(xb_r * wwp_r, axis=1, keepdims=True)
                   + wb_ref[0, 0, 1])
    freq = _sigmoid(wave_f) * (MAX_FREQ - MIN_FREQ) + MIN_FREQ
    phase = _tanh(wave_p) * MAX_FREQ
    kern_h = _silu(_dot_t(xb, kw_ref[...]) + kb_ref[0, 0, :])

    base = pl.multiple_of(jnp.clip(l0 - 128, 0, L - WIN), 128)
    x_win = xh_ref[0, pl.ds(base, WIN), :]

    rel = (lax.broadcasted_iota(jnp.int32, (1, S), 1) - HALF_S).astype(jnp.float32)
    l_col = (l0 + lax.broadcasted_iota(jnp.int32, (BLK, S), 0)).astype(jnp.float32)
    lane_j = lax.broadcasted_iota(jnp.int32, (BLK, WIN), 1)

    off = rel * freq + phase
    off = jnp.clip(off, -MAX_RECEPTIVE, MAX_RECEPTIVE)
    pos = l_col + off
    pos0 = jnp.floor(pos)
    frac = pos - pos0
    i0 = jnp.clip(pos0, 0, L - 1).astype(jnp.int32)
    i1 = jnp.clip(pos0 + 1.0, 0, L - 1).astype(jnp.int32)
    k_idx = jnp.clip(
        jnp.floor((off + MAX_RECEPTIVE) * (K / (2.0 * MAX_RECEPTIVE))),
        0, K - 1).astype(jnp.int32)
    w = jnp.zeros((BLK, S), jnp.float32)
    for k in range(K):
        w = jnp.where(k_idx == k, kern_h[:, k:k + 1], w)
    a0 = w * (1.0 - frac)
    a1 = w * frac
    j0 = i0 - base
    j1 = i1 - base
    acc = jnp.zeros((BLK, WIN), jnp.float32)
    for s in range(S):
        acc += jnp.where(lane_j == j0[:, s:s + 1], a0[:, s:s + 1], 0.0)
        acc += jnp.where(lane_j == j1[:, s:s + 1], a1[:, s:s + 1], 0.0)
    hid_ref[0, :, :] = jnp.dot(acc.astype(jnp.bfloat16),
                               x_win.astype(jnp.bfloat16),
                               preferred_element_type=jnp.float32)


def _out_kernel(hid_ref, ow_ref, o_ref):
    o_ref[...] = _silu(_dot_t(hid_ref[...], ow_ref[...]))


@jax.jit
def kernel(x, wave_w, wave_b, kernel_w, kernel_b, out_w):
    Bi, Li, Ci = x.shape
    x2 = x.reshape(Li, Ci)
    xh = x2.reshape(Li, H, D).transpose(1, 0, 2)
    ww3 = wave_w.reshape(2 * H, 1, Ci)
    wb3 = wave_b.reshape(2, H).T.reshape(H, 1, 2)
    kb3 = kernel_b.reshape(H, 1, K)

    hidden = pl.pallas_call(
        _gather_kernel,
        grid=(Li // BLK, H),
        in_specs=[
            pl.BlockSpec((BLK, Ci), lambda i, h: (i, 0)),
            pl.BlockSpec((1, Li, D), lambda i, h: (h, 0, 0)),
            pl.BlockSpec((1, 1, Ci), lambda i, h: (h, 0, 0)),
            pl.BlockSpec((1, 1, Ci), lambda i, h: (h + H, 0, 0)),
            pl.BlockSpec((1, 1, 2), lambda i, h: (h, 0, 0)),
            pl.BlockSpec((K, Ci), lambda i, h: (h, 0)),
            pl.BlockSpec((1, 1, K), lambda i, h: (h, 0, 0)),
        ],
        out_specs=pl.BlockSpec((1, BLK, D), lambda i, h: (h, i, 0)),
        out_shape=jax.ShapeDtypeStruct((H, Li, D), jnp.float32),
    )(x2, xh, ww3, ww3, wb3, kernel_w, kb3)

    hidden2 = hidden.transpose(1, 0, 2).reshape(Li, Ci)

    out = pl.pallas_call(
        _out_kernel,
        grid=(Li // BLK,),
        in_specs=[
            pl.BlockSpec((BLK, Ci), lambda i: (i, 0)),
            pl.BlockSpec((Ci, Ci), lambda i: (0, 0)),
        ],
        out_specs=pl.BlockSpec((BLK, Ci), lambda i: (i, 0)),
        out_shape=jax.ShapeDtypeStruct((Li, Ci), jnp.float32),
    )(hidden2, out_w)
    return out.reshape(Bi, Li, Ci)


# speedup vs baseline: 27.7037x; 1.3826x over previous
import dataclasses
import functools

import jax
import jax.numpy as jnp
from jax import lax
from jax.experimental import pallas as pl
from jax.experimental.pallas import tpu as pltpu
from jax.experimental.pallas import tpu_sc as plsc

L = 2048
C = 1024
H = 16
K = 64
D = C // H
S = 17
HALF_S = 8
MAX_FREQ = 16.0
MIN_FREQ = 1.0
MAX_RECEPTIVE = 128.0

BLK = 256
CH = 64
W = 392
BASE_MAX = L - W

NLANE = 16


def _sigmoid(t):
    return 1.0 / (1.0 + jnp.exp(-t))


def _tanh(t):
    return 2.0 / (1.0 + jnp.exp(-2.0 * t)) - 1.0


def _silu(t):
    return t * _sigmoid(t)


def _dot_t(a, b):
    return lax.dot_general(
        a.astype(jnp.bfloat16), b.astype(jnp.bfloat16), (((1,), (1,)), ((), ())),
        preferred_element_type=jnp.float32,
    )


def _prep_kernel(xb_ref, wwf_ref, wwp_ref, wb_ref, kw_ref, kb_ref,
                 jr_ref, a0_ref, a1_ref):
    i = pl.program_id(0)
    l0 = i * BLK
    xb = xb_ref[...]

    xb_r = xb.astype(jnp.bfloat16).astype(jnp.float32)
    wwf_r = wwf_ref[0].astype(jnp.bfloat16).astype(jnp.float32)
    wwp_r = wwp_ref[0].astype(jnp.bfloat16).astype(jnp.float32)
    wave_f = _silu(jnp.sum(xb_r * wwf_r, axis=1, keepdims=True)
                   + wb_ref[0, 0, 0])
    wave_p = _silu(jnp.sum(xb_r * wwp_r, axis=1, keepdims=True)
                   + wb_ref[0, 0, 1])
    freq = _sigmoid(wave_f) * (MAX_FREQ - MIN_FREQ) + MIN_FREQ
    phase = _tanh(wave_p) * MAX_FREQ
    kern_h = _silu(_dot_t(xb, kw_ref[...]) + kb_ref[0, 0, :])

    rel = (lax.broadcasted_iota(jnp.int32, (1, S), 1) - HALF_S).astype(jnp.float32)
    li = l0 + lax.broadcasted_iota(jnp.int32, (BLK, S), 0)
    l_col = li.astype(jnp.float32)

    off = rel * freq + phase
    off = jnp.clip(off, -MAX_RECEPTIVE, MAX_RECEPTIVE)
    pos = l_col + off
    pos0 = jnp.floor(pos)
    frac = pos - pos0
    i0 = jnp.clip(pos0, 0, L - 1).astype(jnp.int32)
    k_idx = jnp.clip(
        jnp.floor((off + MAX_RECEPTIVE) * (K / (2.0 * MAX_RECEPTIVE))),
        0, K - 1).astype(jnp.int32)
    w = jnp.zeros((BLK, S), jnp.float32)
    for k in range(K):
        w = jnp.where(k_idx == k, kern_h[:, k:k + 1], w)
    clipped = jnp.logical_or(pos0 < 0.0, pos0 + 1.0 > L - 1)
    a0 = jnp.where(clipped, w, w * (1.0 - frac))
    a1 = jnp.where(clipped, 0.0, w * frac)
    gbase = jnp.clip((li // CH) * CH - 128, 0, BASE_MAX)
    jr_ref[0, :, :] = i0 - gbase
    a0_ref[0, :, :] = a0
    a1_ref[0, :, :] = a1


def _sc_gather_kernel(xh_hbm, jr_hbm, a0_hbm, a1_hbm, hid_hbm,
                      xw, jrv, a0v, a1v, outv):
    wid = 2 * lax.axis_index("s") + lax.axis_index("c")
    lstart = pl.multiple_of(wid * CH, CH)
    base = pl.multiple_of(jnp.clip(wid * CH - 128, 0, BASE_MAX), 8)
    iota = lax.broadcasted_iota(jnp.int32, (NLANE,), 0)

    @pl.loop(0, H)
    def _h_loop(h):
        pltpu.sync_copy(xh_hbm.at[h, pl.ds(base, W), :], xw)
        pltpu.sync_copy(jr_hbm.at[h, pl.ds(lstart, CH), :], jrv)
        pltpu.sync_copy(a0_hbm.at[h, pl.ds(lstart, CH), :], a0v)
        pltpu.sync_copy(a1_hbm.at[h, pl.ds(lstart, CH), :], a1v)

        @pl.loop(0, CH)
        def _row_loop(r):
            rv = jnp.full((NLANE,), r, jnp.int32)
            accs = [jnp.zeros((NLANE,), jnp.float32) for _ in range(D // NLANE)]
            for s in range(S):
                sv = jnp.full((NLANE,), s, jnp.int32)
                j0b = plsc.load_gather(jrv, [rv, sv])
                a0b = plsc.load_gather(a0v, [rv, sv])
                a1b = plsc.load_gather(a1v, [rv, sv])
                j1b = j0b + 1
                for dc in range(D // NLANE):
                    cols = iota + NLANE * dc
                    g0 = plsc.load_gather(xw, [j0b, cols])
                    g1 = plsc.load_gather(xw, [j1b, cols])
                    accs[dc] = accs[dc] + a0b * g0 + a1b * g1
            for dc in range(D // NLANE):
                plsc.store_scatter(outv, [rv, iota + NLANE * dc], accs[dc])

        pltpu.sync_copy(outv, hid_hbm.at[h, pl.ds(lstart, CH), :])


def _out_kernel(hid_ref, ow_ref, o_ref):
    o_ref[...] = _silu(_dot_t(hid_ref[...], ow_ref[...]))


@jax.jit
def kernel(x, wave_w, wave_b, kernel_w, kernel_b, out_w):
    Bi, Li, Ci = x.shape
    x2 = x.reshape(Li, Ci)
    xh = x2.reshape(Li, H, D).transpose(1, 0, 2)
    ww3 = wave_w.reshape(2 * H, 1, Ci)
    wb3 = wave_b.reshape(2, H).T.reshape(H, 1, 2)
    kb3 = kernel_b.reshape(H, 1, K)

    jr, a0, a1 = pl.pallas_call(
        _prep_kernel,
        grid=(Li // BLK, H),
        in_specs=[
            pl.BlockSpec((BLK, Ci), lambda i, h: (i, 0)),
            pl.BlockSpec((1, 1, Ci), lambda i, h: (h, 0, 0)),
            pl.BlockSpec((1, 1, Ci), lambda i, h: (h + H, 0, 0)),
            pl.BlockSpec((1, 1, 2), lambda i, h: (h, 0, 0)),
            pl.BlockSpec((K, Ci), lambda i, h: (h, 0)),
            pl.BlockSpec((1, 1, K), lambda i, h: (h, 0, 0)),
        ],
        out_specs=[
            pl.BlockSpec((1, BLK, S), lambda i, h: (h, i, 0)),
            pl.BlockSpec((1, BLK, S), lambda i, h: (h, i, 0)),
            pl.BlockSpec((1, BLK, S), lambda i, h: (h, i, 0)),
        ],
        out_shape=[
            jax.ShapeDtypeStruct((H, Li, S), jnp.int32),
            jax.ShapeDtypeStruct((H, Li, S), jnp.float32),
            jax.ShapeDtypeStruct((H, Li, S), jnp.float32),
        ],
    )(x2, ww3, ww3, wb3, kernel_w, kb3)

    sc_params = pltpu.CompilerParams()
    if "needs_layout_passes" in pltpu.CompilerParams.__dataclass_fields__:
        sc_params = dataclasses.replace(sc_params, needs_layout_passes=False)
    sc_gather = pl.kernel(
        _sc_gather_kernel,
        out_type=jax.ShapeDtypeStruct((H, Li, D), jnp.float32),
        compiler_params=sc_params,
        mesh=plsc.VectorSubcoreMesh(core_axis_name="c", subcore_axis_name="s"),
        scratch_types=[
            pltpu.VMEM((W, D), jnp.float32),
            pltpu.VMEM((CH, S), jnp.int32),
            pltpu.VMEM((CH, S), jnp.float32),
            pltpu.VMEM((CH, S), jnp.float32),
            pltpu.VMEM((CH, D), jnp.float32),
        ],
    )
    hidden = sc_gather(xh, jr, a0, a1)

    hidden2 = hidden.transpose(1, 0, 2).reshape(Li, Ci)

    out = pl.pallas_call(
        _out_kernel,
        grid=(Li // BLK,),
        in_specs=[
            pl.BlockSpec((BLK, Ci), lambda i: (i, 0)),
            pl.BlockSpec((Ci, Ci), lambda i: (0, 0)),
        ],
        out_specs=pl.BlockSpec((BLK, Ci), lambda i: (i, 0)),
        out_shape=jax.ShapeDtypeStruct((Li, Ci), jnp.float32),
    )(hidden2, out_w)
    return out.reshape(Bi, Li, Ci)
